# baseline (speedup 1.0000x reference)
import functools

import jax
import jax.numpy as jnp
from jax import lax
from jax.experimental import pallas as pl
from jax.experimental.pallas import tpu as pltpu
from jax.experimental.pallas import tpu_sc as plsc

DEPTH = 4
B_TOK = 16384
CODE_DIM = 256
N_CODES = 1024

BLK = 1024


def _vq_body(z_ref, cb_ref, ct_ref, zq_ref, m0_ref, m1_ref, m2_ref, m3_ref):
    r = z_ref[...]
    ct = ct_ref[...]
    cb = cb_ref[...]
    ctn = jnp.sum(ct * ct, axis=1)
    maps_refs = (m0_ref, m1_ref, m2_ref, m3_ref)
    zq = jnp.zeros_like(r)
    for i in range(DEPTH):
        rn = jnp.sum(r * r, axis=1, keepdims=True)
        prod = jax.lax.dot_general(
            r, ct, (((1,), (1,)), ((), ())),
            preferred_element_type=jnp.float32)
        dist = rn + ctn[None, :] - 2.0 * prod
        maps_refs[i][...] = dist
        pred = jnp.argmin(dist, axis=1)
        onehot = (jax.lax.broadcasted_iota(jnp.int32, (BLK, N_CODES), 1)
                  == pred[:, None]).astype(jnp.float32)
        delta = jax.lax.dot_general(
            onehot, cb, (((1,), (0,)), ((), ())),
            preferred_element_type=jnp.float32)
        zq = zq + delta
        r = r - delta
    zq_ref[...] = zq


def _make_sc_gather():
    info = plsc.get_sparse_core_info()
    NC, NS = info.num_cores, info.num_subcores
    NW = NC * NS
    b_per_w = B_TOK // NW
    CH = 256
    n_ch = b_per_w // CH
    mesh = plsc.VectorSubcoreMesh(core_axis_name="c", subcore_axis_name="s")

    @functools.partial(
        pl.kernel, mesh=mesh,
        out_type=jax.ShapeDtypeStruct((B_TOK, CODE_DIM), jnp.float32),
        scratch_types=[
            pltpu.VMEM((CH,), jnp.int32),
            pltpu.VMEM((CH, CODE_DIM), jnp.float32),
            pltpu.SemaphoreType.DMA,
        ],
    )
    def gather_k(table_hbm, idx_hbm, out_hbm, idx_v, rows_v, sem):
        wid = lax.axis_index("s") * NC + lax.axis_index("c")
        for c in range(n_ch):
            base = wid * b_per_w + c * CH
            pltpu.sync_copy(idx_hbm.at[pl.ds(base, CH)], idx_v)
            pltpu.async_copy(table_hbm.at[idx_v], rows_v, sem).wait()
            pltpu.sync_copy(rows_v, out_hbm.at[pl.ds(base, CH)])

    return gather_k


_sc_gather = _make_sc_gather()


@jax.jit
def kernel(z, codebook, codebook_t):
    grid = (B_TOK // BLK,)
    row_block = pl.BlockSpec((BLK, CODE_DIM), lambda i: (i, 0))
    full_cb = pl.BlockSpec((N_CODES, CODE_DIM), lambda i: (0, 0))
    map_block = pl.BlockSpec((BLK, N_CODES), lambda i: (i, 0))
    out_shapes = (
        jax.ShapeDtypeStruct((B_TOK, CODE_DIM), jnp.float32),
        *(jax.ShapeDtypeStruct((B_TOK, N_CODES), jnp.float32),) * DEPTH,
    )
    zq, m0, m1, m2, m3 = pl.pallas_call(
        _vq_body,
        grid=grid,
        in_specs=[row_block, full_cb, full_cb],
        out_specs=(row_block, *(map_block,) * DEPTH),
        out_shape=out_shapes,
        compiler_params=pltpu.CompilerParams(
            dimension_semantics=("parallel",)),
    )(z, codebook, codebook_t)
    idx = jnp.arange(B_TOK, dtype=jnp.int32) % N_CODES
    g = _sc_gather(codebook, idx)
    return (zq + 0.0 * g, m0, m1, m2, m3)


# speedup vs baseline: 1.0901x; 1.0901x over previous
import functools

import jax
import jax.numpy as jnp
from jax.experimental import pallas as pl
from jax.experimental.pallas import tpu as pltpu

DEPTH = 4
B_TOK = 16384
CODE_DIM = 256
N_CODES = 1024

BLK = 1024


def _vq_body(z_ref, cb_ref, ct_ref, zq_ref, m0_ref, m1_ref, m2_ref, m3_ref):
    r = z_ref[...]
    ct = ct_ref[...]
    cb = cb_ref[...]
    ctn = jnp.sum(ct * ct, axis=1)
    maps_refs = (m0_ref, m1_ref, m2_ref, m3_ref)
    iota = jax.lax.broadcasted_iota(jnp.int32, (BLK, N_CODES), 1)
    zq = jnp.zeros_like(r)
    for i in range(DEPTH):
        rn = jnp.sum(r * r, axis=1, keepdims=True)
        prod = jax.lax.dot_general(
            r, ct, (((1,), (1,)), ((), ())),
            preferred_element_type=jnp.float32)
        dist = rn + ctn[None, :] - 2.0 * prod
        maps_refs[i][...] = dist
        minv = jnp.min(dist, axis=1, keepdims=True)
        pred = jnp.min(jnp.where(dist == minv, iota, N_CODES),
                       axis=1, keepdims=True)
        onehot = (iota == pred).astype(jnp.float32)
        delta = jax.lax.dot_general(
            onehot, cb, (((1,), (0,)), ((), ())),
            preferred_element_type=jnp.float32)
        zq = zq + delta
        r = r - delta
    zq_ref[...] = zq


@jax.jit
def kernel(z, codebook, codebook_t):
    grid = (B_TOK // BLK,)
    row_block = pl.BlockSpec((BLK, CODE_DIM), lambda i: (i, 0))
    full_cb = pl.BlockSpec((N_CODES, CODE_DIM), lambda i: (0, 0))
    map_block = pl.BlockSpec((BLK, N_CODES), lambda i: (i, 0))
    out_shapes = (
        jax.ShapeDtypeStruct((B_TOK, CODE_DIM), jnp.float32),
        *(jax.ShapeDtypeStruct((B_TOK, N_CODES), jnp.float32),) * DEPTH,
    )
    zq, m0, m1, m2, m3 = pl.pallas_call(
        _vq_body,
        grid=grid,
        in_specs=[row_block, full_cb, full_cb],
        out_specs=(row_block, *(map_block,) * DEPTH),
        out_shape=out_shapes,
        compiler_params=pltpu.CompilerParams(
            dimension_semantics=("parallel",)),
    )(z, codebook, codebook_t)
    return (zq, m0, m1, m2, m3)
